# Initial kernel scaffold; baseline (speedup 1.0000x reference)
import jax
import jax.numpy as jnp
from jax.experimental import pallas as pl


def kernel(x, edge_index, edge_type, batch, comp1, basis1, root1, rb1, gn1_w, gn1_b, gn1_a, comp2, basis2, root2, rb2, gn2_w, gn2_b, gn2_a, W1, b1, W2, b2):
    raise NotImplementedError("write your pallas kernel here")


# speedup vs baseline: 4.0969x; 4.0969x over previous
import functools

import jax
import jax.numpy as jnp
from jax import lax
from jax.experimental import pallas as pl
from jax.experimental.pallas import tpu as pltpu
from jax.experimental.pallas import tpu_sc as plsc

N = 10000
E = 320000
NPAD = 10240
D3 = 3 * NPAD
G = 16
H = 192
EPS = 1e-5
KE = 80
ROWS_PT = D3 // 16
FB = 384


def _agg_sc(xcat, src, dst, ety, CH, nc):
    mesh = plsc.VectorSubcoreMesh(core_axis_name="c", subcore_axis_name="s",
                                  num_cores=2, num_subcores=16)
    EPC = E // 16
    NIT = EPC // KE
    NZ = ROWS_PT // KE
    zeros_h = jnp.zeros((KE, CH), jnp.float32)

    @functools.partial(
        pl.kernel,
        out_type=jax.ShapeDtypeStruct((nc, D3, CH), jnp.float32),
        mesh=mesh,
        scratch_types=[
            pltpu.VMEM_SHARED((D3, CH), jnp.float32),
            pltpu.VMEM((KE, CH), jnp.float32),
            pltpu.VMEM((KE,), jnp.int32),
            pltpu.VMEM((1, KE), jnp.int32),
            pltpu.VMEM((KE,), jnp.int32),
            pltpu.VMEM((KE,), jnp.int32),
            pltpu.SemaphoreType.DMA,
        ],
        compiler_params=pltpu.CompilerParams(use_tc_tiling_on_sc=False),
    )
    def k(x_hbm, src_hbm, dst_hbm, ety_hbm, z_hbm, out_hbm,
          acc, rows, sidx, didx, dbuf, tbuf, sem):
        c = lax.axis_index("c")
        s = lax.axis_index("s")
        base_e = s * EPC
        row0 = s * ROWS_PT
        for ck in range(nc // 2):
            ci = 2 * ck + c
            pltpu.sync_copy(z_hbm, rows)

            def zbody(z, carry):
                pltpu.sync_copy(rows, acc.at[pl.ds(row0 + z * KE, KE)])
                return carry

            lax.fori_loop(0, NZ, zbody, 0)
            plsc.subcore_barrier()

            def body(kk, carry):
                e0 = base_e + kk * KE
                pltpu.sync_copy(src_hbm.at[pl.ds(e0, KE)], sidx)
                pltpu.sync_copy(dst_hbm.at[pl.ds(e0, KE)], dbuf)
                pltpu.sync_copy(ety_hbm.at[pl.ds(e0, KE)], tbuf)
                for i in range(KE // 16):
                    sl = pl.ds(i * 16, 16)
                    sidx[sl] = sidx[sl] + ci * N
                    didx[0, sl] = tbuf[sl] * NPAD + dbuf[sl]
                pltpu.async_copy(x_hbm.at[sidx], rows, sem).wait()
                pltpu.sync_copy(rows, acc.at[didx.at[0]], add=True)
                return carry

            lax.fori_loop(0, NIT, body, 0)
            plsc.subcore_barrier()

            def fbody(z, carry):
                r0 = row0 + z * KE
                pltpu.sync_copy(acc.at[pl.ds(r0, KE)], rows)
                pltpu.sync_copy(rows, out_hbm.at[ci, pl.ds(r0, KE)])
                return carry

            lax.fori_loop(0, NZ, fbody, 0)
            plsc.subcore_barrier()

    return k(xcat, src, dst, ety, zeros_h)


def _counts_sc(dst, ety):
    mesh = plsc.VectorSubcoreMesh(core_axis_name="c", subcore_axis_name="s",
                                  num_cores=2, num_subcores=16)
    EPT = E // 32
    NIT = EPT // KE
    zeros_h = jnp.zeros((ROWS_PT, 16), jnp.float32)
    ones_h = jnp.ones((KE, 16), jnp.float32)

    @functools.partial(
        pl.kernel,
        out_type=jax.ShapeDtypeStruct((2, D3, 16), jnp.float32),
        mesh=mesh,
        scratch_types=[
            pltpu.VMEM_SHARED((D3, 16), jnp.float32),
            pltpu.VMEM((ROWS_PT, 16), jnp.float32),
            pltpu.VMEM((KE, 16), jnp.float32),
            pltpu.VMEM((1, KE), jnp.int32),
            pltpu.VMEM((KE,), jnp.int32),
            pltpu.VMEM((KE,), jnp.int32),
        ],
        compiler_params=pltpu.CompilerParams(use_tc_tiling_on_sc=False),
    )
    def k(dst_hbm, ety_hbm, z_hbm, ones_hbm, out_hbm,
          acc, fbuf, obuf, didx, dbuf, tbuf):
        c = lax.axis_index("c")
        s = lax.axis_index("s")
        base_e = (c * 16 + s) * EPT
        row0 = s * ROWS_PT
        pltpu.sync_copy(z_hbm, fbuf)
        pltpu.sync_copy(fbuf, acc.at[pl.ds(row0, ROWS_PT)])
        pltpu.sync_copy(ones_hbm, obuf)
        plsc.subcore_barrier()

        def body(kk, carry):
            e0 = base_e + kk * KE
            pltpu.sync_copy(dst_hbm.at[pl.ds(e0, KE)], dbuf)
            pltpu.sync_copy(ety_hbm.at[pl.ds(e0, KE)], tbuf)
            for i in range(KE // 16):
                sl = pl.ds(i * 16, 16)
                didx[0, sl] = tbuf[sl] * NPAD + dbuf[sl]
            pltpu.sync_copy(obuf, acc.at[didx.at[0]], add=True)
            return carry

        lax.fori_loop(0, NIT, body, 0)
        plsc.subcore_barrier()
        pltpu.sync_copy(acc.at[pl.ds(row0, ROWS_PT)], fbuf)
        pltpu.sync_copy(fbuf, out_hbm.at[c, pl.ds(row0, ROWS_PT)])

    return k(dst, ety, zeros_h, ones_h)


BLK = 400
NBLK = N // BLK


def _mkweights(comp, basis):
    _, Cin, Ho = basis.shape

    def body(c_ref, b_ref, o_ref):
        for r in range(3):
            w = c_ref[r, 0] * b_ref[0]
            w = w + c_ref[r, 1] * b_ref[1]
            w = w + c_ref[r, 2] * b_ref[2]
            o_ref[r] = w

    return pl.pallas_call(
        body,
        out_shape=jax.ShapeDtypeStruct((3, Cin, Ho), jnp.float32),
        in_specs=[
            pl.BlockSpec(memory_space=pltpu.MemorySpace.SMEM),
            pl.BlockSpec(memory_space=pltpu.MemorySpace.VMEM),
        ],
        out_specs=pl.BlockSpec(memory_space=pltpu.MemorySpace.VMEM),
    )(comp, basis)


def _linear(xin, agg, cnt, root, wst, rbias, CH, nc):
    Cin = xin.shape[1]
    agg4 = agg.reshape(nc, 3, NPAD, CH)
    cnt4 = cnt.reshape(2, 3, NPAD, 16)
    rb2 = rbias.reshape(1, H)

    def body(x_ref, a_ref, c_ref, root_ref, w_ref, rb_ref, o_ref):
        acc = jnp.dot(x_ref[...], root_ref[...],
                      preferred_element_type=jnp.float32) + rb_ref[...]
        cs = c_ref[0] + c_ref[1]
        for r in range(3):
            scale = 1.0 / jnp.maximum(cs[r, :, 0:1], 1.0)
            for ci in range(nc):
                a = a_ref[ci, r] * scale
                w = w_ref[r, ci * CH:(ci + 1) * CH, :]
                acc = acc + jnp.dot(a, w, preferred_element_type=jnp.float32)
        o_ref[...] = acc

    return pl.pallas_call(
        body,
        grid=(NBLK,),
        out_shape=jax.ShapeDtypeStruct((N, H), jnp.float32),
        in_specs=[
            pl.BlockSpec((BLK, Cin), lambda i: (i, 0)),
            pl.BlockSpec((nc, 3, BLK, CH), lambda i: (0, 0, i, 0)),
            pl.BlockSpec((2, 3, BLK, 16), lambda i: (0, 0, i, 0)),
            pl.BlockSpec((Cin, H), lambda i: (0, 0)),
            pl.BlockSpec((3, Cin, H), lambda i: (0, 0, 0)),
            pl.BlockSpec((1, H), lambda i: (0, 0)),
        ],
        out_specs=pl.BlockSpec((BLK, H), lambda i: (i, 0)),
    )(xin, agg4, cnt4, root, wst, rb2)


def _stats(h, batch2):
    def body(h_ref, b_ref, s_ref, q_ref, c_ref):
        i = pl.program_id(0)

        @pl.when(i == 0)
        def _():
            s_ref[...] = jnp.zeros_like(s_ref)
            q_ref[...] = jnp.zeros_like(q_ref)
            c_ref[...] = jnp.zeros_like(c_ref)

        gi = lax.broadcasted_iota(jnp.int32, (BLK, G), 1)
        oh = (b_ref[...] == gi).astype(jnp.float32)
        hv = h_ref[...]
        dn = (((0,), (0,)), ((), ()))
        s_ref[...] += lax.dot_general(oh, hv, dn,
                                      preferred_element_type=jnp.float32)
        q_ref[...] += lax.dot_general(oh, hv * hv, dn,
                                      preferred_element_type=jnp.float32)
        c_ref[...] += lax.dot_general(oh, jnp.ones((BLK, 8), jnp.float32), dn,
                                      preferred_element_type=jnp.float32)

    return pl.pallas_call(
        body,
        grid=(NBLK,),
        out_shape=[
            jax.ShapeDtypeStruct((G, H), jnp.float32),
            jax.ShapeDtypeStruct((G, H), jnp.float32),
            jax.ShapeDtypeStruct((G, 8), jnp.float32),
        ],
        in_specs=[
            pl.BlockSpec((BLK, H), lambda i: (i, 0)),
            pl.BlockSpec((BLK, 1), lambda i: (i, 0)),
        ],
        out_specs=[
            pl.BlockSpec((G, H), lambda i: (0, 0)),
            pl.BlockSpec((G, H), lambda i: (0, 0)),
            pl.BlockSpec((G, 8), lambda i: (0, 0)),
        ],
    )(h, batch2)


def _apply_norm(h, batch2, sums, sumsq, cntg, w, b, a):
    w2, b2, a2 = w.reshape(1, H), b.reshape(1, H), a.reshape(1, H)

    def body(h_ref, bt_ref, s_ref, q_ref, c_ref, w_ref, bb_ref, a_ref, o_ref):
        cg = jnp.maximum(c_ref[:, 0:1], 1.0)
        mean = s_ref[...] / cg
        av = a_ref[...]
        var = q_ref[...] / cg - 2.0 * av * mean * (s_ref[...] / cg) \
            + av * av * mean * mean
        inv = 1.0 / jnp.sqrt(var + EPS)
        gi = lax.broadcasted_iota(jnp.int32, (BLK, G), 1)
        oh = (bt_ref[...] == gi).astype(jnp.float32)
        mb = jnp.dot(oh, av * mean, preferred_element_type=jnp.float32)
        ib = jnp.dot(oh, inv, preferred_element_type=jnp.float32)
        o = (h_ref[...] - mb) * ib * w_ref[...] + bb_ref[...]
        o_ref[...] = jnp.maximum(o, 0.0)

    return pl.pallas_call(
        body,
        grid=(NBLK,),
        out_shape=jax.ShapeDtypeStruct((N, H), jnp.float32),
        in_specs=[
            pl.BlockSpec((BLK, H), lambda i: (i, 0)),
            pl.BlockSpec((BLK, 1), lambda i: (i, 0)),
            pl.BlockSpec((G, H), lambda i: (0, 0)),
            pl.BlockSpec((G, H), lambda i: (0, 0)),
            pl.BlockSpec((G, 8), lambda i: (0, 0)),
            pl.BlockSpec((1, H), lambda i: (0, 0)),
            pl.BlockSpec((1, H), lambda i: (0, 0)),
            pl.BlockSpec((1, H), lambda i: (0, 0)),
        ],
        out_specs=pl.BlockSpec((BLK, H), lambda i: (i, 0)),
    )(h, batch2, sums, sumsq, cntg, w2, b2, a2)


def _head(sums, cntg, W1, b1, W2, b2):
    b1r, b2r = b1.reshape(1, -1), b2.reshape(1, -1)

    def body(s_ref, c_ref, w1_ref, b1_ref, w2_ref, b2_ref, o_ref):
        g = s_ref[...] / jnp.maximum(c_ref[:, 0:1], 1.0)
        t = jnp.dot(g, w1_ref[...], preferred_element_type=jnp.float32) \
            + b1_ref[...]
        t = jnp.maximum(t, 0.0)
        o_ref[...] = jnp.dot(t, w2_ref[...],
                             preferred_element_type=jnp.float32) + b2_ref[...]

    return pl.pallas_call(
        body,
        out_shape=jax.ShapeDtypeStruct((G, 4), jnp.float32),
    )(sums, cntg, W1, b1r, W2, b2r)


def kernel(x, edge_index, edge_type, batch, comp1, basis1, root1, rb1,
           gn1_w, gn1_b, gn1_a, comp2, basis2, root2, rb2, gn2_w, gn2_b,
           gn2_a, W1, b1, W2, b2):
    src = edge_index[0].astype(jnp.int32)
    dst = edge_index[1].astype(jnp.int32)
    ety = edge_type.astype(jnp.int32)
    batch2 = batch.astype(jnp.int32).reshape(N, 1)

    cnt = _counts_sc(dst, ety)
    wst1 = _mkweights(comp1, basis1)
    wst2 = _mkweights(comp2, basis2)

    xcat = x.reshape(N, 2, 64).transpose(1, 0, 2).reshape(2 * N, 64)
    agg1 = _agg_sc(xcat, src, dst, ety, CH=64, nc=2)
    h1p = _linear(x, agg1, cnt, root1, wst1, rb1, CH=64, nc=2)
    s1, q1, cg = _stats(h1p, batch2)
    h1 = _apply_norm(h1p, batch2, s1, q1, cg, gn1_w, gn1_b, gn1_a)

    hcat = h1.reshape(N, 4, 48).transpose(1, 0, 2).reshape(4 * N, 48)
    agg2 = _agg_sc(hcat, src, dst, ety, CH=48, nc=4)
    h2p = _linear(h1, agg2, cnt, root2, wst2, rb2, CH=48, nc=4)
    s2, q2, _ = _stats(h2p, batch2)
    h2 = _apply_norm(h2p, batch2, s2, q2, cg, gn2_w, gn2_b, gn2_a)

    sp, _, _ = _stats(h2, batch2)
    return _head(sp, cg, W1, b1, W2, b2)


# speedup vs baseline: 7.3954x; 1.8051x over previous
import functools

import jax
import jax.numpy as jnp
from jax import lax
from jax.experimental import pallas as pl
from jax.experimental.pallas import tpu as pltpu
from jax.experimental.pallas import tpu_sc as plsc

N = 10000
E = 320000
NPAD = 10240
D3 = 3 * NPAD
G = 16
H = 192
EPS = 1e-5
KE = 80
ER = E // KE
NSB = 25
ROWS_PT = D3 // 16
FL = 160

_SC_MESH = dict(core_axis_name="c", subcore_axis_name="s",
                num_cores=2, num_subcores=16)
_SC_PARAMS = pltpu.CompilerParams(use_tc_tiling_on_sc=False)


def _agg_sc(xcat, sidx_all, didx_all, CH, nc):
    RPT = ER // 16
    NBLK2 = RPT // (2 * NSB)
    zeros_h = jnp.zeros((FL, CH), jnp.float32)

    @functools.partial(
        pl.kernel,
        out_type=jax.ShapeDtypeStruct((nc, D3, CH), jnp.float32),
        mesh=plsc.VectorSubcoreMesh(**_SC_MESH),
        scratch_types=[
            pltpu.VMEM_SHARED((D3, CH), jnp.float32),
            pltpu.VMEM((KE, CH), jnp.float32),
            pltpu.VMEM((KE, CH), jnp.float32),
            pltpu.VMEM((NSB, KE), jnp.int32),
            pltpu.VMEM((NSB, KE), jnp.int32),
            pltpu.VMEM((NSB, KE), jnp.int32),
            pltpu.VMEM((NSB, KE), jnp.int32),
            pltpu.VMEM((FL, CH), jnp.float32),
            pltpu.VMEM((FL, CH), jnp.float32),
            pltpu.SemaphoreType.DMA,
            pltpu.SemaphoreType.DMA,
            pltpu.SemaphoreType.DMA,
            pltpu.SemaphoreType.DMA,
            pltpu.SemaphoreType.DMA,
        ],
        compiler_params=_SC_PARAMS,
    )
    def k(x_hbm, sidx_hbm, didx_hbm, z_hbm, out_hbm,
          acc, rows0, rows1, sA, sB, dA, dB, fl0, fl1,
          gsem, ssem, eA, eB, fsem):
        c = lax.axis_index("c")
        s = lax.axis_index("s")
        erow0 = s * RPT
        row0 = s * ROWS_PT
        rows = (rows0, rows1)
        fls = (fl0, fl1)

        def do_block(sbuf, dbuf):
            pltpu.async_copy(x_hbm.at[sbuf.at[0]], rows0, gsem)
            for j in range(NSB):
                rj = rows[j % 2]
                pltpu.make_async_copy(x_hbm.at[sbuf.at[j]], rj, gsem).wait()
                if j + 1 < NSB:
                    if j >= 1:
                        pltpu.make_async_copy(
                            rows[(j + 1) % 2],
                            acc.at[dbuf.at[j - 1]], ssem).wait()
                    pltpu.async_copy(x_hbm.at[sbuf.at[j + 1]],
                                     rows[(j + 1) % 2], gsem)
                pltpu.async_copy(rj, acc.at[dbuf.at[j]], ssem, add=True)
            pltpu.make_async_copy(rows0, acc.at[dbuf.at[0]], ssem).wait()
            pltpu.make_async_copy(rows1, acc.at[dbuf.at[0]], ssem).wait()

        def load_block(ci, b, sbuf, dbuf, sem):
            r0 = erow0 + b * NSB
            pltpu.async_copy(sidx_hbm.at[ci, pl.ds(r0, NSB)], sbuf, sem)
            pltpu.async_copy(didx_hbm.at[pl.ds(r0, NSB)], dbuf, sem)

        def wait_block(sbuf, dbuf, sem):
            dummy = didx_hbm.at[pl.ds(0, NSB)]
            pltpu.make_async_copy(dummy, sbuf, sem).wait()
            pltpu.make_async_copy(dummy, dbuf, sem).wait()

        def pass_body(ck, carry):
            ci = 2 * ck + c
            pltpu.sync_copy(z_hbm, fl0)
            for z in range(ROWS_PT // FL):
                pltpu.async_copy(fl0, acc.at[pl.ds(row0 + z * FL, FL)], fsem)
            for z in range(ROWS_PT // FL):
                pltpu.make_async_copy(fl0, acc.at[pl.ds(row0, FL)],
                                      fsem).wait()
            plsc.subcore_barrier()

            load_block(ci, 0, sA, dA, eA)

            def blk_pair(g, carry2):
                load_block(ci, 2 * g + 1, sB, dB, eB)
                wait_block(sA, dA, eA)
                do_block(sA, dA)

                @pl.when(g + 1 < NBLK2)
                def _():
                    load_block(ci, 2 * g + 2, sA, dA, eA)

                wait_block(sB, dB, eB)
                do_block(sB, dB)
                return carry2

            lax.fori_loop(0, NBLK2, blk_pair, 0)
            plsc.subcore_barrier()

            pltpu.async_copy(acc.at[pl.ds(row0, FL)], fl0, fsem)
            for z in range(ROWS_PT // FL):
                fz = fls[z % 2]
                pltpu.make_async_copy(acc.at[pl.ds(row0, FL)], fz,
                                      fsem).wait()
                if z + 1 < ROWS_PT // FL:
                    pltpu.async_copy(
                        acc.at[pl.ds(row0 + (z + 1) * FL, FL)],
                        fls[(z + 1) % 2], fsem)
                pltpu.sync_copy(fz, out_hbm.at[ci, pl.ds(row0 + z * FL, FL)])
            plsc.subcore_barrier()
            return carry

        lax.fori_loop(0, nc // 2, pass_body, 0)

    return k(xcat, sidx_all, didx_all, zeros_h)


def _counts_sc(didx_all):
    RPT = ER // 32
    NB = RPT // NSB
    zeros_h = jnp.zeros((ROWS_PT, 16), jnp.float32)
    ones_h = jnp.ones((KE, 16), jnp.float32)

    @functools.partial(
        pl.kernel,
        out_type=jax.ShapeDtypeStruct((2, D3, 16), jnp.float32),
        mesh=plsc.VectorSubcoreMesh(**_SC_MESH),
        scratch_types=[
            pltpu.VMEM_SHARED((D3, 16), jnp.float32),
            pltpu.VMEM((ROWS_PT, 16), jnp.float32),
            pltpu.VMEM((KE, 16), jnp.float32),
            pltpu.VMEM((NSB, KE), jnp.int32),
            pltpu.VMEM((NSB, KE), jnp.int32),
            pltpu.SemaphoreType.DMA,
            pltpu.SemaphoreType.DMA,
        ],
        compiler_params=_SC_PARAMS,
    )
    def k(didx_hbm, z_hbm, ones_hbm, out_hbm,
          acc, fbuf, obuf, dA, dB, esem, ssem):
        c = lax.axis_index("c")
        s = lax.axis_index("s")
        erow0 = (c * 16 + s) * RPT
        row0 = s * ROWS_PT
        pltpu.sync_copy(z_hbm, fbuf)
        pltpu.sync_copy(fbuf, acc.at[pl.ds(row0, ROWS_PT)])
        pltpu.sync_copy(ones_hbm, obuf)
        plsc.subcore_barrier()

        bufs = (dA, dB)
        pltpu.async_copy(didx_hbm.at[pl.ds(erow0, NSB)], dA, esem)

        for b in range(NB):
            dbuf = bufs[b % 2]
            pltpu.make_async_copy(didx_hbm.at[pl.ds(0, NSB)], dbuf,
                                  esem).wait()
            if b + 1 < NB:
                pltpu.async_copy(
                    didx_hbm.at[pl.ds(erow0 + (b + 1) * NSB, NSB)],
                    bufs[(b + 1) % 2], esem)
            for j in range(NSB):
                pltpu.async_copy(obuf, acc.at[dbuf.at[j]], ssem, add=True)
            for j in range(NSB):
                pltpu.make_async_copy(obuf, acc.at[dbuf.at[0]], ssem).wait()
        plsc.subcore_barrier()
        pltpu.sync_copy(acc.at[pl.ds(row0, ROWS_PT)], fbuf)
        pltpu.sync_copy(fbuf, out_hbm.at[c, pl.ds(row0, ROWS_PT)])

    return k(didx_all, zeros_h, ones_h)


BLK = 400
NBLK = N // BLK


def _mkidx(src, dst, ety, nc):
    s2 = src.reshape(ER, KE)
    d2 = dst.reshape(ER, KE)
    t2 = ety.reshape(ER, KE)
    RB = 400

    def body(s_ref, d_ref, t_ref, os_ref, od_ref):
        sv = s_ref[...]
        od_ref[...] = t_ref[...] * NPAD + d_ref[...]
        for ci in range(nc):
            os_ref[ci] = sv + ci * N

    return pl.pallas_call(
        body,
        grid=(ER // RB,),
        out_shape=[
            jax.ShapeDtypeStruct((nc, ER, KE), jnp.int32),
            jax.ShapeDtypeStruct((ER, KE), jnp.int32),
        ],
        in_specs=[
            pl.BlockSpec((RB, KE), lambda i: (i, 0)),
            pl.BlockSpec((RB, KE), lambda i: (i, 0)),
            pl.BlockSpec((RB, KE), lambda i: (i, 0)),
        ],
        out_specs=[
            pl.BlockSpec((nc, RB, KE), lambda i: (0, i, 0)),
            pl.BlockSpec((RB, KE), lambda i: (i, 0)),
        ],
    )(s2, d2, t2)


def _mkweights(comp, basis):
    _, Cin, Ho = basis.shape

    def body(c_ref, b_ref, o_ref):
        for r in range(3):
            w = c_ref[r, 0] * b_ref[0]
            w = w + c_ref[r, 1] * b_ref[1]
            w = w + c_ref[r, 2] * b_ref[2]
            o_ref[r] = w

    return pl.pallas_call(
        body,
        out_shape=jax.ShapeDtypeStruct((3, Cin, Ho), jnp.float32),
        in_specs=[
            pl.BlockSpec(memory_space=pltpu.MemorySpace.SMEM),
            pl.BlockSpec(memory_space=pltpu.MemorySpace.VMEM),
        ],
        out_specs=pl.BlockSpec(memory_space=pltpu.MemorySpace.VMEM),
    )(comp, basis)


def _linear(xin, agg, cnt, root, wst, rbias, CH, nc):
    Cin = xin.shape[1]
    agg4 = agg.reshape(nc, 3, NPAD, CH)
    cnt4 = cnt.reshape(2, 3, NPAD, 16)
    rb2 = rbias.reshape(1, H)

    def body(x_ref, a_ref, c_ref, root_ref, w_ref, rb_ref, o_ref):
        acc = jnp.dot(x_ref[...], root_ref[...],
                      preferred_element_type=jnp.float32) + rb_ref[...]
        cs = c_ref[0] + c_ref[1]
        for r in range(3):
            scale = 1.0 / jnp.maximum(cs[r, :, 0:1], 1.0)
            for ci in range(nc):
                a = a_ref[ci, r] * scale
                w = w_ref[r, ci * CH:(ci + 1) * CH, :]
                acc = acc + jnp.dot(a, w, preferred_element_type=jnp.float32)
        o_ref[...] = acc

    return pl.pallas_call(
        body,
        grid=(NBLK,),
        out_shape=jax.ShapeDtypeStruct((N, H), jnp.float32),
        in_specs=[
            pl.BlockSpec((BLK, Cin), lambda i: (i, 0)),
            pl.BlockSpec((nc, 3, BLK, CH), lambda i: (0, 0, i, 0)),
            pl.BlockSpec((2, 3, BLK, 16), lambda i: (0, 0, i, 0)),
            pl.BlockSpec((Cin, H), lambda i: (0, 0)),
            pl.BlockSpec((3, Cin, H), lambda i: (0, 0, 0)),
            pl.BlockSpec((1, H), lambda i: (0, 0)),
        ],
        out_specs=pl.BlockSpec((BLK, H), lambda i: (i, 0)),
    )(xin, agg4, cnt4, root, wst, rb2)


def _stats(h, batch2):
    def body(h_ref, b_ref, s_ref, q_ref, c_ref):
        i = pl.program_id(0)

        @pl.when(i == 0)
        def _():
            s_ref[...] = jnp.zeros_like(s_ref)
            q_ref[...] = jnp.zeros_like(q_ref)
            c_ref[...] = jnp.zeros_like(c_ref)

        gi = lax.broadcasted_iota(jnp.int32, (BLK, G), 1)
        oh = (b_ref[...] == gi).astype(jnp.float32)
        hv = h_ref[...]
        dn = (((0,), (0,)), ((), ()))
        s_ref[...] += lax.dot_general(oh, hv, dn,
                                      preferred_element_type=jnp.float32)
        q_ref[...] += lax.dot_general(oh, hv * hv, dn,
                                      preferred_element_type=jnp.float32)
        c_ref[...] += lax.dot_general(oh, jnp.ones((BLK, 8), jnp.float32), dn,
                                      preferred_element_type=jnp.float32)

    return pl.pallas_call(
        body,
        grid=(NBLK,),
        out_shape=[
            jax.ShapeDtypeStruct((G, H), jnp.float32),
            jax.ShapeDtypeStruct((G, H), jnp.float32),
            jax.ShapeDtypeStruct((G, 8), jnp.float32),
        ],
        in_specs=[
            pl.BlockSpec((BLK, H), lambda i: (i, 0)),
            pl.BlockSpec((BLK, 1), lambda i: (i, 0)),
        ],
        out_specs=[
            pl.BlockSpec((G, H), lambda i: (0, 0)),
            pl.BlockSpec((G, H), lambda i: (0, 0)),
            pl.BlockSpec((G, 8), lambda i: (0, 0)),
        ],
    )(h, batch2)


def _apply_norm(h, batch2, sums, sumsq, cntg, w, b, a):
    w2, b2, a2 = w.reshape(1, H), b.reshape(1, H), a.reshape(1, H)

    def body(h_ref, bt_ref, s_ref, q_ref, c_ref, w_ref, bb_ref, a_ref, o_ref):
        cg = jnp.maximum(c_ref[:, 0:1], 1.0)
        mean = s_ref[...] / cg
        av = a_ref[...]
        var = q_ref[...] / cg - 2.0 * av * mean * (s_ref[...] / cg) \
            + av * av * mean * mean
        inv = 1.0 / jnp.sqrt(var + EPS)
        gi = lax.broadcasted_iota(jnp.int32, (BLK, G), 1)
        oh = (bt_ref[...] == gi).astype(jnp.float32)
        mb = jnp.dot(oh, av * mean, preferred_element_type=jnp.float32)
        ib = jnp.dot(oh, inv, preferred_element_type=jnp.float32)
        o = (h_ref[...] - mb) * ib * w_ref[...] + bb_ref[...]
        o_ref[...] = jnp.maximum(o, 0.0)

    return pl.pallas_call(
        body,
        grid=(NBLK,),
        out_shape=jax.ShapeDtypeStruct((N, H), jnp.float32),
        in_specs=[
            pl.BlockSpec((BLK, H), lambda i: (i, 0)),
            pl.BlockSpec((BLK, 1), lambda i: (i, 0)),
            pl.BlockSpec((G, H), lambda i: (0, 0)),
            pl.BlockSpec((G, H), lambda i: (0, 0)),
            pl.BlockSpec((G, 8), lambda i: (0, 0)),
            pl.BlockSpec((1, H), lambda i: (0, 0)),
            pl.BlockSpec((1, H), lambda i: (0, 0)),
            pl.BlockSpec((1, H), lambda i: (0, 0)),
        ],
        out_specs=pl.BlockSpec((BLK, H), lambda i: (i, 0)),
    )(h, batch2, sums, sumsq, cntg, w2, b2, a2)


def _head(sums, cntg, W1, b1, W2, b2):
    b1r, b2r = b1.reshape(1, -1), b2.reshape(1, -1)

    def body(s_ref, c_ref, w1_ref, b1_ref, w2_ref, b2_ref, o_ref):
        g = s_ref[...] / jnp.maximum(c_ref[:, 0:1], 1.0)
        t = jnp.dot(g, w1_ref[...], preferred_element_type=jnp.float32) \
            + b1_ref[...]
        t = jnp.maximum(t, 0.0)
        o_ref[...] = jnp.dot(t, w2_ref[...],
                             preferred_element_type=jnp.float32) + b2_ref[...]

    return pl.pallas_call(
        body,
        out_shape=jax.ShapeDtypeStruct((G, 4), jnp.float32),
    )(sums, cntg, W1, b1r, W2, b2r)


def kernel(x, edge_index, edge_type, batch, comp1, basis1, root1, rb1,
           gn1_w, gn1_b, gn1_a, comp2, basis2, root2, rb2, gn2_w, gn2_b,
           gn2_a, W1, b1, W2, b2):
    src = edge_index[0].astype(jnp.int32)
    dst = edge_index[1].astype(jnp.int32)
    ety = edge_type.astype(jnp.int32)
    batch2 = batch.astype(jnp.int32).reshape(N, 1)

    sidx4, didx = _mkidx(src, dst, ety, nc=4)
    cnt = _counts_sc(didx)
    wst1 = _mkweights(comp1, basis1)
    wst2 = _mkweights(comp2, basis2)

    xcat = x.reshape(N, 4, 32).transpose(1, 0, 2).reshape(4 * N, 32)
    agg1 = _agg_sc(xcat, sidx4, didx, CH=32, nc=4)
    h1p = _linear(x, agg1, cnt, root1, wst1, rb1, CH=32, nc=4)
    s1, q1, cg = _stats(h1p, batch2)
    h1 = _apply_norm(h1p, batch2, s1, q1, cg, gn1_w, gn1_b, gn1_a)

    hcat = h1.reshape(N, 4, 48).transpose(1, 0, 2).reshape(4 * N, 48)
    agg2 = _agg_sc(hcat, sidx4, didx, CH=48, nc=4)
    h2p = _linear(h1, agg2, cnt, root2, wst2, rb2, CH=48, nc=4)
    s2, q2, _ = _stats(h2p, batch2)
    h2 = _apply_norm(h2p, batch2, s2, q2, cg, gn2_w, gn2_b, gn2_a)

    sp, _, _ = _stats(h2, batch2)
    return _head(sp, cg, W1, b1, W2, b2)
